# baseline (speedup 1.0000x reference)
import functools

import jax
import jax.numpy as jnp
from jax import lax
from jax.experimental import pallas as pl
from jax.experimental.pallas import tpu as pltpu
from jax.experimental.pallas import tpu_sc as plsc

NCORE = 2
NSUB = 16
NW = NCORE * NSUB
CHUNK = 128


def _cdiv(a, b):
    return (a + b - 1) // b


def _make_agg(NA, F, EPW):
    CPW = EPW // CHUNK
    assert CPW % 4 == 0 and CPW >= 4
    RPS = NA // NSUB
    mesh = plsc.VectorSubcoreMesh(core_axis_name="c", subcore_axis_name="s")

    @functools.partial(
        pl.kernel,
        out_type=jax.ShapeDtypeStruct((NCORE * NA, F), jnp.float32),
        mesh=mesh,
        scratch_types=(
            [pltpu.VMEM_SHARED((NA, F), jnp.float32)]
            + [pltpu.VMEM((CHUNK,), jnp.int32)] * 8
            + [pltpu.VMEM((CHUNK, F), jnp.float32)] * 2
            + [pltpu.SemaphoreType.DMA] * 7
        ),
    )
    def agg(y_hbm, src_hbm, dst_hbm, zz_hbm, out_hbm, acc, *scr):
        sidx = scr[0:4]
        didx = scr[4:8]
        rows = scr[8:10]
        semI = scr[10:14]
        semG = scr[14]
        semS = scr[15:17]
        cid = lax.axis_index("c")
        sid = lax.axis_index("s")
        wid = cid * NSUB + sid

        for u in range(4):
            base = wid * EPW + u * CHUNK
            pltpu.async_copy(src_hbm.at[pl.ds(base, CHUNK)], sidx[u], semI[u])
            pltpu.async_copy(dst_hbm.at[pl.ds(base, CHUNK)], didx[u], semI[u])

        pltpu.sync_copy(zz_hbm.at[pl.ds(sid * RPS, RPS)],
                        acc.at[pl.ds(sid * RPS, RPS)])
        plsc.subcore_barrier()

        @pl.loop(0, CPW // 4)
        def _(jj):
            for u in range(4):
                b2 = u % 2
                bw = (u + 2) % 4
                j = jj * 4 + u

                def wait_scatter(b2=b2, bw=bw):
                    pltpu.make_async_copy(
                        rows[b2], acc.at[didx[bw]], semS[b2]).wait()

                def prefetch(j=j, bw=bw):
                    nbase = wid * EPW + (j + 2) * CHUNK
                    pltpu.async_copy(
                        src_hbm.at[pl.ds(nbase, CHUNK)], sidx[bw], semI[bw])
                    pltpu.async_copy(
                        dst_hbm.at[pl.ds(nbase, CHUNK)], didx[bw], semI[bw])

                def wait_and_prefetch():
                    wait_scatter()
                    prefetch()

                if u < 2:
                    pl.when(jj > 0)(wait_and_prefetch)
                else:
                    wait_scatter()
                    pl.when(jj < CPW // 4 - 1)(prefetch)

                base = wid * EPW + j * CHUNK
                pltpu.make_async_copy(
                    src_hbm.at[pl.ds(base, CHUNK)], sidx[u], semI[u]).wait()
                pltpu.make_async_copy(
                    dst_hbm.at[pl.ds(base, CHUNK)], didx[u], semI[u]).wait()
                pltpu.async_copy(y_hbm.at[sidx[u]], rows[b2], semG).wait()
                pltpu.async_copy(rows[b2], acc.at[didx[u]], semS[b2], add=True)

        pltpu.make_async_copy(rows[0], acc.at[didx[(CPW - 2) % 4]],
                              semS[0]).wait()
        pltpu.make_async_copy(rows[1], acc.at[didx[(CPW - 1) % 4]],
                              semS[1]).wait()
        plsc.subcore_barrier()
        pltpu.sync_copy(acc.at[pl.ds(sid * RPS, RPS)],
                        out_hbm.at[pl.ds(cid * NA + sid * RPS, RPS)])

    return agg


def _pre_body(x_ref, w1b_ref, yaug_ref):
    x = x_ref[...]
    y = jnp.dot(x, w1b_ref[...], preferred_element_type=jnp.float32)
    cols = lax.broadcasted_iota(jnp.int32, (x.shape[0], 64), 1)
    extra = jnp.where(cols == 0, 1.0, 0.0).astype(jnp.float32)
    yaug_ref[...] = jnp.concatenate([y, extra], axis=1)


def _mid_body(n, a0_ref, a1_ref, x_ref, w1a_ref, b1_ref, w2a_ref, w2b_ref,
              z_ref, h1a_ref, rdeg_ref):
    s = a0_ref[0:n, 0:64] + a1_ref[0:n, 0:64]
    deg = a0_ref[0:n, 64:65] + a1_ref[0:n, 64:65]
    rdeg = 1.0 / jnp.maximum(deg, 1.0)
    xa = jnp.dot(x_ref[...], w1a_ref[...], preferred_element_type=jnp.float32)
    h1 = jnp.maximum(xa + s * rdeg + b1_ref[...], 0.0)
    z = jnp.dot(h1, w2b_ref[...], preferred_element_type=jnp.float32)
    z_ref[...] = jnp.pad(z, ((0, 0), (0, 96)))
    h1a_ref[...] = jnp.dot(h1, w2a_ref[...], preferred_element_type=jnp.float32)
    rdeg_ref[...] = rdeg


def _post_body(n, a0_ref, a1_ref, h1a_ref, rdeg_ref, b2_ref, w3_ref, b3_ref,
               out_ref):
    s2 = a0_ref[0:n, 0:32] + a1_ref[0:n, 0:32]
    mean2 = s2 * rdeg_ref[...]
    h2 = jnp.maximum(h1a_ref[...] + mean2 + b2_ref[...], 0.0)
    out_ref[...] = (jnp.dot(h2, w3_ref[...], preferred_element_type=jnp.float32)
                    + b3_ref[...])


def kernel(x, edge_index, W1, b1, W2, b2, W3, b3):
    N, D = x.shape
    E = edge_index.shape[1]
    F1 = W1.shape[1]
    F2 = W2.shape[1]

    EPW = _cdiv(E, NW * CHUNK * 4) * CHUNK * 4
    E_pad = EPW * NW
    NA = _cdiv(N + 1, NSUB * 8) * NSUB * 8
    trash = NA - N

    src = edge_index[0]
    dst = edge_index[1]
    pad_e = E_pad - E
    src = jnp.concatenate(
        [src, jnp.zeros((pad_e + 2 * CHUNK,), jnp.int32)])
    dst = jnp.concatenate(
        [dst, N + (jnp.arange(pad_e, dtype=jnp.int32) % trash),
         jnp.zeros((2 * CHUNK,), jnp.int32)])

    W1a, W1b = W1[:D], W1[D:]
    W2a, W2b = W2[:F1], W2[F1:]
    FA = 128
    zz = jnp.zeros((NA, FA), jnp.float32)

    yaug = pl.pallas_call(
        _pre_body,
        out_shape=jax.ShapeDtypeStruct((N, FA), jnp.float32),
    )(x, W1b)

    agg1 = _make_agg(NA, FA, EPW)
    r1 = agg1(yaug, src, dst, zz)
    a10, a11 = r1[:NA], r1[NA:]

    z, h1a, rdeg = pl.pallas_call(
        functools.partial(_mid_body, N),
        out_shape=(
            jax.ShapeDtypeStruct((N, FA), jnp.float32),
            jax.ShapeDtypeStruct((N, F2), jnp.float32),
            jax.ShapeDtypeStruct((N, 1), jnp.float32),
        ),
    )(a10, a11, x, W1a, b1.reshape(1, F1), W2a, W2b)

    agg2 = _make_agg(NA, FA, EPW)
    r2 = agg2(z, src, dst, zz)
    a20, a21 = r2[:NA], r2[NA:]

    out = pl.pallas_call(
        functools.partial(_post_body, N),
        out_shape=jax.ShapeDtypeStruct((N, 1), jnp.float32),
    )(a20, a21, h1a, rdeg, b2.reshape(1, F2), W3, b3.reshape(1, 1))

    return out


# speedup vs baseline: 5.2326x; 5.2326x over previous
import functools

import jax
import jax.numpy as jnp
from jax import lax
from jax.experimental import pallas as pl
from jax.experimental.pallas import tpu as pltpu
from jax.experimental.pallas import tpu_sc as plsc

NCORE = 2
NSUB = 16
NW = NCORE * NSUB
CHUNK = 128


def _cdiv(a, b):
    return (a + b - 1) // b


def _make_agg(NA, F, EPW, K=2):
    CPW = EPW // CHUNK
    G = CPW // K
    assert CPW % K == 0 and G % 2 == 0 and G >= 2
    RPS = NA // NSUB
    NB = 2 * K
    mesh = plsc.VectorSubcoreMesh(core_axis_name="c", subcore_axis_name="s")

    @functools.partial(
        pl.kernel,
        out_type=jax.ShapeDtypeStruct((NCORE * NA, F), jnp.float32),
        mesh=mesh,
        compiler_params=pltpu.CompilerParams(use_tc_tiling_on_sc=False),
        scratch_types=(
            [pltpu.VMEM_SHARED((NA, F), jnp.float32)]
            + [pltpu.VMEM((CPW, CHUNK), jnp.int32)] * 2
            + [pltpu.VMEM((CHUNK, F), jnp.float32)] * NB
            + [pltpu.SemaphoreType.DMA] * 4
        ),
    )
    def agg(y_hbm, src_hbm, dst_hbm, zz_hbm, out_hbm, acc, *scr):
        sidx, didx = scr[0], scr[1]
        rows = scr[2:2 + NB]
        semG = scr[2 + NB:4 + NB]
        semS = scr[4 + NB:6 + NB]
        cid = lax.axis_index("c")
        sid = lax.axis_index("s")
        wid = cid * NSUB + sid

        pltpu.sync_copy(src_hbm.at[wid], sidx)
        pltpu.sync_copy(dst_hbm.at[wid], didx)
        pltpu.sync_copy(zz_hbm.at[pl.ds(sid * RPS, RPS)],
                        acc.at[pl.ds(sid * RPS, RPS)])
        for i in range(K):
            pltpu.async_copy(y_hbm.at[sidx.at[i]], rows[i], semG[0])
        plsc.subcore_barrier()

        def body(g, a):
            b = 1 - a
            for i in range(K):
                pltpu.make_async_copy(
                    y_hbm.at[sidx.at[0]], rows[a * K + i], semG[a]).wait()

            @pl.when(g >= 1)
            def _():
                for i in range(K):
                    pltpu.make_async_copy(
                        rows[b * K + i], acc.at[didx.at[0]], semS[b]).wait()

            @pl.when(g + 1 < G)
            def _():
                for i in range(K):
                    pltpu.async_copy(
                        y_hbm.at[sidx.at[(g + 1) * K + i]], rows[b * K + i],
                        semG[b])

            for i in range(K):
                pltpu.async_copy(
                    rows[a * K + i], acc.at[didx.at[g * K + i]], semS[a],
                    add=True)

        @pl.loop(0, G // 2)
        def _(t):
            body(2 * t, 0)
            body(2 * t + 1, 1)

        for i in range(K):
            pltpu.make_async_copy(
                rows[K + i], acc.at[didx.at[0]], semS[1]).wait()
        plsc.subcore_barrier()
        pltpu.sync_copy(acc.at[pl.ds(sid * RPS, RPS)],
                        out_hbm.at[pl.ds(cid * NA + sid * RPS, RPS)])

    return agg


def _pre_body(x_ref, w1b_ref, yaug_ref):
    x = x_ref[...]
    y = jnp.dot(x, w1b_ref[...], preferred_element_type=jnp.float32)
    cols = lax.broadcasted_iota(jnp.int32, (x.shape[0], 16), 1)
    extra = jnp.where(cols == 0, 1.0, 0.0).astype(jnp.float32)
    yaug_ref[...] = jnp.concatenate([y, extra], axis=1)


def _mid_body(n, a0_ref, a1_ref, x_ref, w1a_ref, b1_ref, w2a_ref, w2b_ref,
              z_ref, h1a_ref, rdeg_ref):
    s = a0_ref[0:n, 0:64] + a1_ref[0:n, 0:64]
    deg = a0_ref[0:n, 64:65] + a1_ref[0:n, 64:65]
    rdeg = 1.0 / jnp.maximum(deg, 1.0)
    xa = jnp.dot(x_ref[...], w1a_ref[...], preferred_element_type=jnp.float32)
    h1 = jnp.maximum(xa + s * rdeg + b1_ref[...], 0.0)
    z_ref[...] = jnp.dot(h1, w2b_ref[...], preferred_element_type=jnp.float32)
    h1a_ref[...] = jnp.dot(h1, w2a_ref[...], preferred_element_type=jnp.float32)
    rdeg_ref[...] = rdeg


def _post_body(n, a0_ref, a1_ref, h1a_ref, rdeg_ref, b2_ref, w3_ref, b3_ref,
               out_ref):
    s2 = a0_ref[0:n, 0:32] + a1_ref[0:n, 0:32]
    mean2 = s2 * rdeg_ref[...]
    h2 = jnp.maximum(h1a_ref[...] + mean2 + b2_ref[...], 0.0)
    out_ref[...] = (jnp.dot(h2, w3_ref[...], preferred_element_type=jnp.float32)
                    + b3_ref[...])


def kernel(x, edge_index, W1, b1, W2, b2, W3, b3):
    N, D = x.shape
    E = edge_index.shape[1]
    F1 = W1.shape[1]
    F2 = W2.shape[1]

    K = 2
    EPW = _cdiv(E, NW * CHUNK * 2 * K) * CHUNK * 2 * K
    E_pad = EPW * NW
    CPW = EPW // CHUNK
    NA = _cdiv(N + 1, NSUB * 8) * NSUB * 8
    trash = NA - N

    pad_e = E_pad - E
    pad_i = jnp.arange(pad_e, dtype=jnp.int32)
    src = jnp.concatenate([edge_index[0], pad_i % N]).reshape(NW, CPW, CHUNK)
    dst = jnp.concatenate(
        [edge_index[1], N + pad_i % trash]).reshape(NW, CPW, CHUNK)

    W1a, W1b = W1[:D], W1[D:]
    W2a, W2b = W2[:F1], W2[F1:]
    FA = F1 + 16
    zz1 = jnp.zeros((NA, FA), jnp.float32)
    zz2 = jnp.zeros((NA, F2), jnp.float32)

    yaug = pl.pallas_call(
        _pre_body,
        out_shape=jax.ShapeDtypeStruct((N, FA), jnp.float32),
    )(x, W1b)

    agg1 = _make_agg(NA, FA, EPW, K)
    r1 = agg1(yaug, src, dst, zz1)
    a10, a11 = r1[:NA], r1[NA:]

    z, h1a, rdeg = pl.pallas_call(
        functools.partial(_mid_body, N),
        out_shape=(
            jax.ShapeDtypeStruct((N, F2), jnp.float32),
            jax.ShapeDtypeStruct((N, F2), jnp.float32),
            jax.ShapeDtypeStruct((N, 1), jnp.float32),
        ),
    )(a10, a11, x, W1a, b1.reshape(1, F1), W2a, W2b)

    agg2 = _make_agg(NA, F2, EPW, K)
    r2 = agg2(z, src, dst, zz2)
    a20, a21 = r2[:NA], r2[NA:]

    out = pl.pallas_call(
        functools.partial(_post_body, N),
        out_shape=jax.ShapeDtypeStruct((N, 1), jnp.float32),
    )(a20, a21, h1a, rdeg, b2.reshape(1, F2), W3, b3.reshape(1, 1))

    return out


# speedup vs baseline: 6.1440x; 1.1742x over previous
import functools

import jax
import jax.numpy as jnp
from jax import lax
from jax.experimental import pallas as pl
from jax.experimental.pallas import tpu as pltpu
from jax.experimental.pallas import tpu_sc as plsc

NCORE = 2
NSUB = 16
NW = NCORE * NSUB
CHUNK = 128


def _cdiv(a, b):
    return (a + b - 1) // b


def _make_agg(N, NA, F, CPW, K):
    G = CPW // K
    assert CPW % K == 0 and G % 2 == 0 and G >= 2
    RPS = NA // NSUB
    NB = 2 * K
    trash = NA - N
    mesh = plsc.VectorSubcoreMesh(core_axis_name="c", subcore_axis_name="s")

    @functools.partial(
        pl.kernel,
        out_type=jax.ShapeDtypeStruct((NCORE * NA, F), jnp.float32),
        mesh=mesh,
        compiler_params=pltpu.CompilerParams(use_tc_tiling_on_sc=False),
        scratch_types=(
            [pltpu.VMEM_SHARED((NA, F), jnp.float32)]
            + [pltpu.VMEM((CPW, CHUNK), jnp.int32)] * 2
            + [pltpu.VMEM((CHUNK, F), jnp.float32)] * NB
            + [pltpu.SemaphoreType.DMA] * 4
        ),
    )
    def agg(ei_hbm, y_hbm, zz_hbm, out_hbm, acc, *scr):
        R = ei_hbm.shape[1]
        LAST = R - (NW - 1) * CPW
        sidx, didx = scr[0], scr[1]
        rows = scr[2:2 + NB]
        semG = scr[2 + NB:4 + NB]
        semS = scr[4 + NB:6 + NB]
        cid = lax.axis_index("c")
        sid = lax.axis_index("s")
        wid = cid * NSUB + sid

        @pl.when(wid < NW - 1)
        def _():
            pltpu.sync_copy(ei_hbm.at[0, pl.ds(wid * CPW, CPW)], sidx)
            pltpu.sync_copy(ei_hbm.at[1, pl.ds(wid * CPW, CPW)], didx)

        @pl.when(wid == NW - 1)
        def _():
            pltpu.sync_copy(ei_hbm.at[0, pl.ds(wid * CPW, LAST)],
                            sidx.at[pl.ds(0, LAST)])
            pltpu.sync_copy(ei_hbm.at[1, pl.ds(wid * CPW, LAST)],
                            didx.at[pl.ds(0, LAST)])
            lanes = lax.iota(jnp.int32, 16)
            PC = CHUNK // 16

            @pl.loop(0, (CPW - LAST) * PC)
            def _(t):
                r = LAST + t // PC
                c = (t % PC) * 16
                g = t * 16 + lanes
                sidx[r, pl.ds(c, 16)] = lax.rem(g, N)
                didx[r, pl.ds(c, 16)] = N + lax.rem(g, trash)

        pltpu.sync_copy(zz_hbm.at[pl.ds(sid * RPS, RPS)],
                        acc.at[pl.ds(sid * RPS, RPS)])
        for i in range(K):
            pltpu.async_copy(y_hbm.at[sidx.at[i]], rows[i], semG[0])
        plsc.subcore_barrier()

        def body(g, a):
            b = 1 - a
            for i in range(K):
                pltpu.make_async_copy(
                    y_hbm.at[sidx.at[0]], rows[a * K + i], semG[a]).wait()

            @pl.when(g >= 1)
            def _():
                for i in range(K):
                    pltpu.make_async_copy(
                        rows[b * K + i], acc.at[didx.at[0]], semS[b]).wait()

            @pl.when(g + 1 < G)
            def _():
                for i in range(K):
                    pltpu.async_copy(
                        y_hbm.at[sidx.at[(g + 1) * K + i]], rows[b * K + i],
                        semG[b])

            for i in range(K):
                pltpu.async_copy(
                    rows[a * K + i], acc.at[didx.at[g * K + i]], semS[a],
                    add=True)

        @pl.loop(0, G // 2)
        def _(t):
            body(2 * t, 0)
            body(2 * t + 1, 1)

        for i in range(K):
            pltpu.make_async_copy(
                rows[K + i], acc.at[didx.at[0]], semS[1]).wait()
        plsc.subcore_barrier()
        pltpu.sync_copy(acc.at[pl.ds(sid * RPS, RPS)],
                        out_hbm.at[pl.ds(cid * NA + sid * RPS, RPS)])

    return agg


def _pre_body(x_ref, w1b_ref, yaug_ref):
    x = x_ref[...]
    y = jnp.dot(x, w1b_ref[...], preferred_element_type=jnp.float32)
    cols = lax.broadcasted_iota(jnp.int32, (x.shape[0], 16), 1)
    extra = jnp.where(cols == 0, 1.0, 0.0).astype(jnp.float32)
    yaug_ref[...] = jnp.concatenate([y, extra], axis=1)


def _mid_body(n, na, a_ref, x_ref, w1a_ref, b1_ref, w2a_ref, w2b_ref,
              z_ref, h1a_ref, rdeg_ref):
    s = a_ref[0:n, 0:64] + a_ref[na:na + n, 0:64]
    deg = a_ref[0:n, 64:65] + a_ref[na:na + n, 64:65]
    rdeg = 1.0 / jnp.maximum(deg, 1.0)
    xa = jnp.dot(x_ref[...], w1a_ref[...], preferred_element_type=jnp.float32)
    h1 = jnp.maximum(xa + s * rdeg + b1_ref[...], 0.0)
    z_ref[...] = jnp.dot(h1, w2b_ref[...], preferred_element_type=jnp.float32)
    h1a_ref[...] = jnp.dot(h1, w2a_ref[...], preferred_element_type=jnp.float32)
    rdeg_ref[...] = rdeg


def _post_body(n, na, a_ref, h1a_ref, rdeg_ref, b2_ref, w3_ref, b3_ref,
               out_ref):
    s2 = a_ref[0:n, :] + a_ref[na:na + n, :]
    mean2 = s2 * rdeg_ref[...]
    h2 = jnp.maximum(h1a_ref[...] + mean2 + b2_ref[...], 0.0)
    out_ref[...] = (jnp.dot(h2, w3_ref[...], preferred_element_type=jnp.float32)
                    + b3_ref[...])


def kernel(x, edge_index, W1, b1, W2, b2, W3, b3):
    N, D = x.shape
    E = edge_index.shape[1]
    F1 = W1.shape[1]
    F2 = W2.shape[1]
    assert E % CHUNK == 0
    R = E // CHUNK

    CPW = _cdiv(R, NW * 4) * 4
    NA = _cdiv(N + 1, NSUB * 8) * NSUB * 8

    ei = edge_index.reshape(2, R, CHUNK)

    W1a, W1b = W1[:D], W1[D:]
    W2a, W2b = W2[:F1], W2[F1:]
    FA = F1 + 16

    zz1 = jnp.zeros((NA, FA), jnp.float32)
    zz2 = jnp.zeros((NA, F2), jnp.float32)

    yaug = pl.pallas_call(
        _pre_body,
        out_shape=jax.ShapeDtypeStruct((N, FA), jnp.float32),
    )(x, W1b)

    r1 = _make_agg(N, NA, FA, CPW, 2)(ei, yaug, zz1)

    z, h1a, rdeg = pl.pallas_call(
        functools.partial(_mid_body, N, NA),
        out_shape=(
            jax.ShapeDtypeStruct((N, F2), jnp.float32),
            jax.ShapeDtypeStruct((N, F2), jnp.float32),
            jax.ShapeDtypeStruct((N, 1), jnp.float32),
        ),
    )(r1, x, W1a, b1.reshape(1, F1), W2a, W2b)

    r2 = _make_agg(N, NA, F2, CPW, 4)(ei, z, zz2)

    out = pl.pallas_call(
        functools.partial(_post_body, N, NA),
        out_shape=jax.ShapeDtypeStruct((N, 1), jnp.float32),
    )(r2, h1a, rdeg, b2.reshape(1, F2), W3, b3.reshape(1, 1))

    return out


# speedup vs baseline: 6.1471x; 1.0005x over previous
import functools

import jax
import jax.numpy as jnp
from jax import lax
from jax.experimental import pallas as pl
from jax.experimental.pallas import tpu as pltpu
from jax.experimental.pallas import tpu_sc as plsc

NCORE = 2
NSUB = 16
NW = NCORE * NSUB
KCMAX = 512


def _cdiv(a, b):
    return (a + b - 1) // b


def _make_agg(N, NA, F, GPW, KC):
    assert GPW % 2 == 0 and GPW >= 2
    RPS = NA // NSUB
    trash = NA - N
    mesh = plsc.VectorSubcoreMesh(core_axis_name="c", subcore_axis_name="s")

    @functools.partial(
        pl.kernel,
        out_type=jax.ShapeDtypeStruct((NCORE * NA, F), jnp.float32),
        mesh=mesh,
        compiler_params=pltpu.CompilerParams(use_tc_tiling_on_sc=False),
        scratch_types=(
            [pltpu.VMEM_SHARED((NA, F), jnp.float32)]
            + [pltpu.VMEM((GPW, KC), jnp.int32)] * 2
            + [pltpu.VMEM((KC, F), jnp.float32)] * 2
            + [pltpu.SemaphoreType.DMA] * 4
        ),
    )
    def agg(ei_hbm, y_hbm, zz_hbm, out_hbm, acc, *scr):
        RG = ei_hbm.shape[1]
        LAST = RG - (NW - 1) * GPW
        sidx, didx = scr[0], scr[1]
        rows = scr[2:4]
        semG = scr[4:6]
        semS = scr[6:8]
        cid = lax.axis_index("c")
        sid = lax.axis_index("s")
        wid = cid * NSUB + sid

        @pl.when(wid < NW - 1)
        def _():
            pltpu.sync_copy(ei_hbm.at[0, pl.ds(wid * GPW, GPW)], sidx)
            pltpu.sync_copy(ei_hbm.at[1, pl.ds(wid * GPW, GPW)], didx)

        @pl.when(wid == NW - 1)
        def _():
            pltpu.sync_copy(ei_hbm.at[0, pl.ds(wid * GPW, LAST)],
                            sidx.at[pl.ds(0, LAST)])
            pltpu.sync_copy(ei_hbm.at[1, pl.ds(wid * GPW, LAST)],
                            didx.at[pl.ds(0, LAST)])
            lanes = lax.iota(jnp.int32, 16)
            PC = KC // 16

            @pl.loop(0, (GPW - LAST) * PC)
            def _(t):
                r = LAST + t // PC
                c = (t % PC) * 16
                g = t * 16 + lanes
                sidx[r, pl.ds(c, 16)] = lax.rem(g, N)
                didx[r, pl.ds(c, 16)] = N + lax.rem(g, trash)

        pltpu.sync_copy(zz_hbm.at[pl.ds(sid * RPS, RPS)],
                        acc.at[pl.ds(sid * RPS, RPS)])
        pltpu.async_copy(y_hbm.at[sidx.at[0]], rows[0], semG[0])
        plsc.subcore_barrier()

        def body(g, a):
            b = 1 - a
            pltpu.make_async_copy(
                y_hbm.at[sidx.at[0]], rows[a], semG[a]).wait()

            @pl.when(g >= 1)
            def _():
                pltpu.make_async_copy(
                    rows[b], acc.at[didx.at[0]], semS[b]).wait()

            @pl.when(g + 1 < GPW)
            def _():
                pltpu.async_copy(y_hbm.at[sidx.at[g + 1]], rows[b], semG[b])

            pltpu.async_copy(
                rows[a], acc.at[didx.at[g]], semS[a], add=True)

        @pl.loop(0, GPW // 2)
        def _(t):
            body(2 * t, 0)
            body(2 * t + 1, 1)

        pltpu.make_async_copy(
            rows[1], acc.at[didx.at[0]], semS[1]).wait()
        plsc.subcore_barrier()
        pltpu.sync_copy(acc.at[pl.ds(sid * RPS, RPS)],
                        out_hbm.at[pl.ds(cid * NA + sid * RPS, RPS)])

    return agg


def _pre_body(x_ref, w1b_ref, yaug_ref):
    x = x_ref[...]
    y = jnp.dot(x, w1b_ref[...], preferred_element_type=jnp.float32)
    cols = lax.broadcasted_iota(jnp.int32, (x.shape[0], 16), 1)
    extra = jnp.where(cols == 0, 1.0, 0.0).astype(jnp.float32)
    yaug_ref[...] = jnp.concatenate([y, extra], axis=1)


def _mid_body(n, na, a_ref, x_ref, w1a_ref, b1_ref, w2a_ref, w2b_ref,
              z_ref, h1a_ref, rdeg_ref):
    s = a_ref[0:n, 0:64] + a_ref[na:na + n, 0:64]
    deg = a_ref[0:n, 64:65] + a_ref[na:na + n, 64:65]
    rdeg = 1.0 / jnp.maximum(deg, 1.0)
    xa = jnp.dot(x_ref[...], w1a_ref[...], preferred_element_type=jnp.float32)
    h1 = jnp.maximum(xa + s * rdeg + b1_ref[...], 0.0)
    z_ref[...] = jnp.dot(h1, w2b_ref[...], preferred_element_type=jnp.float32)
    h1a_ref[...] = jnp.dot(h1, w2a_ref[...], preferred_element_type=jnp.float32)
    rdeg_ref[...] = rdeg


def _post_body(n, na, a_ref, h1a_ref, rdeg_ref, b2_ref, w3_ref, b3_ref,
               out_ref):
    s2 = a_ref[0:n, :] + a_ref[na:na + n, :]
    mean2 = s2 * rdeg_ref[...]
    h2 = jnp.maximum(h1a_ref[...] + mean2 + b2_ref[...], 0.0)
    out_ref[...] = (jnp.dot(h2, w3_ref[...], preferred_element_type=jnp.float32)
                    + b3_ref[...])


def kernel(x, edge_index, W1, b1, W2, b2, W3, b3):
    N, D = x.shape
    E = edge_index.shape[1]
    F1 = W1.shape[1]
    F2 = W2.shape[1]
    KC1, KC2 = 256, 512
    assert E % KC1 == 0 and E % KC2 == 0
    GPW1 = _cdiv(E // KC1, NW * 2) * 2
    GPW2 = _cdiv(E // KC2, NW * 2) * 2
    NA = _cdiv(N + 1, NSUB * 8) * NSUB * 8

    ei1 = edge_index.reshape(2, E // KC1, KC1)
    ei2 = edge_index.reshape(2, E // KC2, KC2)

    W1a, W1b = W1[:D], W1[D:]
    W2a, W2b = W2[:F1], W2[F1:]
    FA = F1 + 16

    zz1 = jnp.zeros((NA, FA), jnp.float32)
    zz2 = jnp.zeros((NA, F2), jnp.float32)

    yaug = pl.pallas_call(
        _pre_body,
        out_shape=jax.ShapeDtypeStruct((N, FA), jnp.float32),
    )(x, W1b)

    r1 = _make_agg(N, NA, FA, GPW1, KC1)(ei1, yaug, zz1)

    z, h1a, rdeg = pl.pallas_call(
        functools.partial(_mid_body, N, NA),
        out_shape=(
            jax.ShapeDtypeStruct((N, F2), jnp.float32),
            jax.ShapeDtypeStruct((N, F2), jnp.float32),
            jax.ShapeDtypeStruct((N, 1), jnp.float32),
        ),
    )(r1, x, W1a, b1.reshape(1, F1), W2a, W2b)

    r2 = _make_agg(N, NA, F2, GPW2, KC2)(ei2, z, zz2)

    out = pl.pallas_call(
        functools.partial(_post_body, N, NA),
        out_shape=jax.ShapeDtypeStruct((N, 1), jnp.float32),
    )(r2, h1a, rdeg, b2.reshape(1, F2), W3, b3.reshape(1, 1))

    return out


# speedup vs baseline: 6.1963x; 1.0080x over previous
import functools

import jax
import jax.numpy as jnp
from jax import lax
from jax.experimental import pallas as pl
from jax.experimental.pallas import tpu as pltpu
from jax.experimental.pallas import tpu_sc as plsc

NCORE = 2
NSUB = 16
NW = NCORE * NSUB
KCMAX = 512


def _cdiv(a, b):
    return (a + b - 1) // b


def _make_agg(N, NA, F, GPW, KC):
    assert GPW % 2 == 0 and GPW >= 2
    RPS = NA // NSUB
    trash = NA - N
    mesh = plsc.VectorSubcoreMesh(core_axis_name="c", subcore_axis_name="s")

    @functools.partial(
        pl.kernel,
        out_type=jax.ShapeDtypeStruct((NCORE * NA, F), jnp.float32),
        mesh=mesh,
        compiler_params=pltpu.CompilerParams(use_tc_tiling_on_sc=False),
        scratch_types=(
            [pltpu.VMEM_SHARED((NA, F), jnp.float32)]
            + [pltpu.VMEM((GPW, KC), jnp.int32)] * 2
            + [pltpu.VMEM((KC, F), jnp.float32)] * 2
            + [pltpu.SemaphoreType.DMA] * 4
        ),
    )
    def agg(ei_hbm, y_hbm, zz_hbm, out_hbm, acc, *scr):
        RG = ei_hbm.shape[1]
        LAST = RG - (NW - 1) * GPW
        sidx, didx = scr[0], scr[1]
        rows = scr[2:4]
        semG = scr[4:6]
        semS = scr[6:8]
        cid = lax.axis_index("c")
        sid = lax.axis_index("s")
        wid = cid * NSUB + sid

        @pl.when(wid < NW - 1)
        def _():
            pltpu.sync_copy(ei_hbm.at[0, pl.ds(wid * GPW, GPW)], sidx)
            pltpu.sync_copy(ei_hbm.at[1, pl.ds(wid * GPW, GPW)], didx)

        @pl.when(wid == NW - 1)
        def _():
            pltpu.sync_copy(ei_hbm.at[0, pl.ds(wid * GPW, LAST)],
                            sidx.at[pl.ds(0, LAST)])
            pltpu.sync_copy(ei_hbm.at[1, pl.ds(wid * GPW, LAST)],
                            didx.at[pl.ds(0, LAST)])
            lanes = lax.iota(jnp.int32, 16)
            PC = KC // 16

            @pl.loop(0, (GPW - LAST) * PC)
            def _(t):
                r = LAST + t // PC
                c = (t % PC) * 16
                g = t * 16 + lanes
                sidx[r, pl.ds(c, 16)] = lax.rem(g, N)
                didx[r, pl.ds(c, 16)] = N + lax.rem(g, trash)

        pltpu.sync_copy(zz_hbm.at[pl.ds(sid * RPS, RPS)],
                        acc.at[pl.ds(sid * RPS, RPS)])
        pltpu.async_copy(y_hbm.at[sidx.at[0]], rows[0], semG[0])
        plsc.subcore_barrier()

        def body(g, a):
            b = 1 - a
            pltpu.make_async_copy(
                y_hbm.at[sidx.at[0]], rows[a], semG[a]).wait()

            @pl.when(g >= 1)
            def _():
                pltpu.make_async_copy(
                    rows[b], acc.at[didx.at[0]], semS[b]).wait()

            @pl.when(g + 1 < GPW)
            def _():
                pltpu.async_copy(y_hbm.at[sidx.at[g + 1]], rows[b], semG[b])

            pltpu.async_copy(
                rows[a], acc.at[didx.at[g]], semS[a], add=True)

        @pl.loop(0, GPW // 2)
        def _(t):
            body(2 * t, 0)
            body(2 * t + 1, 1)

        pltpu.make_async_copy(
            rows[1], acc.at[didx.at[0]], semS[1]).wait()
        plsc.subcore_barrier()
        pltpu.sync_copy(acc.at[pl.ds(sid * RPS, RPS)],
                        out_hbm.at[pl.ds(cid * NA + sid * RPS, RPS)])

    return agg


def _pre_body(d, pad, x_ref, w1_ref, yaug_ref):
    x = x_ref[...]
    y = jnp.dot(x, w1_ref[d:2 * d, :], preferred_element_type=jnp.float32)
    cols = lax.broadcasted_iota(jnp.int32, (x.shape[0], pad), 1)
    extra = jnp.where(cols == 0, 1.0, 0.0).astype(jnp.float32)
    yaug_ref[...] = jnp.concatenate([y, extra], axis=1)


def _mid_body(n, na, d, f1, a_ref, x_ref, w1_ref, b1_ref, w2_ref,
              z_ref, h1a_ref, rdeg_ref):
    s = a_ref[0:n, 0:f1] + a_ref[na:na + n, 0:f1]
    deg = a_ref[0:n, f1:f1 + 1] + a_ref[na:na + n, f1:f1 + 1]
    rdeg = 1.0 / jnp.maximum(deg, 1.0)
    xa = jnp.dot(x_ref[...], w1_ref[0:d, :], preferred_element_type=jnp.float32)
    h1 = jnp.maximum(xa + s * rdeg + b1_ref[...].reshape(1, -1), 0.0)
    z_ref[...] = jnp.dot(h1, w2_ref[f1:2 * f1, :],
                         preferred_element_type=jnp.float32)
    h1a_ref[...] = jnp.dot(h1, w2_ref[0:f1, :],
                           preferred_element_type=jnp.float32)
    rdeg_ref[...] = rdeg


def _post_body(n, na, a_ref, h1a_ref, rdeg_ref, b2_ref, w3_ref, b3_ref,
               out_ref):
    s2 = a_ref[0:n, :] + a_ref[na:na + n, :]
    mean2 = s2 * rdeg_ref[...]
    h2 = jnp.maximum(h1a_ref[...] + mean2 + b2_ref[...].reshape(1, -1), 0.0)
    out_ref[...] = (jnp.dot(h2, w3_ref[...], preferred_element_type=jnp.float32)
                    + b3_ref[...].reshape(1, 1))


def kernel(x, edge_index, W1, b1, W2, b2, W3, b3):
    N, D = x.shape
    E = edge_index.shape[1]
    F1 = W1.shape[1]
    F2 = W2.shape[1]
    KC1, KC2 = 256, 512
    assert E % KC1 == 0 and E % KC2 == 0
    GPW1 = _cdiv(E // KC1, NW * 2) * 2
    GPW2 = _cdiv(E // KC2, NW * 2) * 2
    NA = _cdiv(N + 1, NSUB * 8) * NSUB * 8

    ei1 = edge_index.reshape(2, E // KC1, KC1)
    ei2 = edge_index.reshape(2, E // KC2, KC2)

    FA = F1 + 8

    zz1 = jnp.zeros((NA, FA), jnp.float32)
    zz2 = jnp.zeros((NA, F2), jnp.float32)

    yaug = pl.pallas_call(
        functools.partial(_pre_body, D, FA - F1),
        out_shape=jax.ShapeDtypeStruct((N, FA), jnp.float32),
    )(x, W1)

    r1 = _make_agg(N, NA, FA, GPW1, KC1)(ei1, yaug, zz1)

    z, h1a, rdeg = pl.pallas_call(
        functools.partial(_mid_body, N, NA, D, F1),
        out_shape=(
            jax.ShapeDtypeStruct((N, F2), jnp.float32),
            jax.ShapeDtypeStruct((N, F2), jnp.float32),
            jax.ShapeDtypeStruct((N, 1), jnp.float32),
        ),
    )(r1, x, W1, b1, W2)

    r2 = _make_agg(N, NA, F2, GPW2, KC2)(ei2, z, zz2)

    out = pl.pallas_call(
        functools.partial(_post_body, N, NA),
        out_shape=jax.ShapeDtypeStruct((N, 1), jnp.float32),
    )(r2, h1a, rdeg, b2, W3, b3)

    return out
